# baseline (speedup 1.0000x reference)
import jax
import jax.numpy as jnp
from jax.experimental import pallas as pl
from jax.experimental.pallas import tpu as pltpu

_N = 50000
_D = 512
_G = 128
_C = 128
_B = 1000
_NB = _N // _B
_NCORE = 2
_NBPC = _NB // _NCORE
_STEPS = (_NBPC + 1) // 2
_NC = 5
_W = _B // _NC


def _epilogue(bounds_ref, batch_ref, h_ref, pooled_ref, jblk, live):
    bb = batch_ref[...]
    for c in range(_NC):
        hc = h_ref[c * _W:(c + 1) * _W, :]
        bbc = bb[c * _W:(c + 1) * _W, :]
        idx = jblk * _NC + c
        s0 = bounds_ref[idx, 0]
        s1 = bounds_ref[idx, 1]
        mask = bbc == s0
        m0 = jnp.max(jnp.where(mask, hc, -jnp.inf), axis=0, keepdims=True)
        m1 = jnp.max(jnp.where(bbc == s1, hc, -jnp.inf), axis=0,
                     keepdims=True)
        g0 = jnp.where(live, s0, _G)
        g1 = jnp.where(live, s1, _G)
        cur0 = pooled_ref[pl.ds(g0, 1), :]
        pooled_ref[pl.ds(g0, 1), :] = jnp.maximum(cur0, m0)
        cur1 = pooled_ref[pl.ds(g1, 1), :]
        pooled_ref[pl.ds(g1, 1), :] = jnp.maximum(cur1, m1)

        @pl.when(jnp.logical_and(live, s1 > s0 + 1))
        def _middles(hc=hc, bbc=bbc, s0=s0, s1=s1):
            def body(g, carry):
                m = jnp.max(jnp.where(bbc == g, hc, -jnp.inf), axis=0,
                            keepdims=True)
                cur = pooled_ref[pl.ds(g, 1), :]
                pooled_ref[pl.ds(g, 1), :] = jnp.maximum(cur, m)
                return carry

            jax.lax.fori_loop(s0 + 1, s1, body, 0)


def _pool_kernel(bounds_ref, xa_ref, xb_ref, batcha_ref, batchb_ref,
                 w_enc_ref, out_ref, pooled_ref, buf_a, buf_b):
    ci = pl.program_id(0)
    j = pl.program_id(1)
    base = ci * _NBPC

    @pl.when(j == 0)
    def _init():
        pooled_ref[...] = jnp.full_like(pooled_ref, -jnp.inf)

    buf_a[...] = jnp.dot(xa_ref[...], w_enc_ref[...],
                         preferred_element_type=jnp.float32)
    jb = base + jnp.maximum(2 * j - 1, 0)
    _epilogue(bounds_ref, batchb_ref, buf_b, pooled_ref, jb, j > 0)

    buf_b[...] = jnp.dot(xb_ref[...], w_enc_ref[...],
                         preferred_element_type=jnp.float32)
    ja = base + jnp.minimum(2 * j, _NBPC - 1)
    _epilogue(bounds_ref, batcha_ref, buf_a, pooled_ref, ja, j >= 0)

    @pl.when(j == _STEPS - 1)
    def _finish():
        out_ref[...] = pooled_ref[0:_G, :]


def _head_kernel(part_ref, b_enc_ref, w_out_ref, b_out_ref, out_ref):
    pooled = jnp.maximum(part_ref[0:_G, :], part_ref[_G:2 * _G, :])
    pooled = jnp.maximum(pooled + b_enc_ref[...], 0.0)
    logits = (
        jnp.dot(pooled, w_out_ref[...], preferred_element_type=jnp.float32)
        + b_out_ref[...]
    )
    mx = jnp.max(logits, axis=1, keepdims=True)
    sh = logits - mx
    lse = jnp.log(jnp.sum(jnp.exp(sh), axis=1, keepdims=True))
    out_ref[...] = sh - lse


def kernel(x, batch, W_enc, b_enc, W_out, b_out):
    batch = batch.astype(jnp.int32)
    batch_col = batch.reshape(_N, 1)
    starts = jnp.arange(_NB * _NC, dtype=jnp.int32) * _W
    bounds = jnp.stack([batch[starts], batch[starts + _W - 1]], axis=1)

    nbm1 = _NB - 1

    grid_spec = pltpu.PrefetchScalarGridSpec(
        num_scalar_prefetch=1,
        grid=(_NCORE, _STEPS),
        in_specs=[
            pl.BlockSpec((_B, _D),
                         lambda c, j, b: (
                             jnp.minimum(c * _NBPC + 2 * j, nbm1), 0)),
            pl.BlockSpec((_B, _D),
                         lambda c, j, b: (
                             jnp.minimum(c * _NBPC + 2 * j + 1, nbm1), 0)),
            pl.BlockSpec((_B, 1),
                         lambda c, j, b: (
                             jnp.minimum(c * _NBPC + 2 * j, nbm1), 0)),
            pl.BlockSpec((_B, 1),
                         lambda c, j, b: (
                             c * _NBPC + jnp.maximum(2 * j - 1, 0), 0)),
            pl.BlockSpec((_D, _D), lambda c, j, b: (0, 0)),
        ],
        out_specs=pl.BlockSpec((_G, _D), lambda c, j, b: (c, 0)),
        scratch_shapes=[pltpu.VMEM((_G + 8, _D), jnp.float32),
                        pltpu.VMEM((_B, _D), jnp.float32),
                        pltpu.VMEM((_B, _D), jnp.float32)],
    )

    partial = pl.pallas_call(
        _pool_kernel,
        grid_spec=grid_spec,
        out_shape=jax.ShapeDtypeStruct((_NCORE * _G, _D), jnp.float32),
        compiler_params=pltpu.CompilerParams(
            dimension_semantics=("parallel", "arbitrary"),
        ),
    )(bounds, x, x, batch_col, batch_col, W_enc)

    return pl.pallas_call(
        _head_kernel,
        out_shape=jax.ShapeDtypeStruct((_G, _C), jnp.float32),
    )(partial, b_enc.reshape(1, _D), W_out, b_out.reshape(1, _C))


# speedup vs baseline: 1.0471x; 1.0471x over previous
import jax
import jax.numpy as jnp
from jax.experimental import pallas as pl
from jax.experimental.pallas import tpu as pltpu

_N = 50000
_D = 512
_G = 128
_C = 128
_B = 1000
_NB = _N // _B
_NB2 = _NB // 2
_NC = 5
_W = _B // _NC


def _epilogue(bounds_ref, batch_ref, h_ref, pooled_ref, jblk, live):
    bb = batch_ref[...]
    for c in range(_NC):
        hc = h_ref[c * _W:(c + 1) * _W, :]
        bbc = bb[c * _W:(c + 1) * _W, :]
        idx = jblk * _NC + c
        s0 = bounds_ref[idx, 0]
        s1 = bounds_ref[idx, 1]
        g0 = jnp.where(live, s0, _G)

        @pl.when(s0 == s1)
        def _pure(hc=hc, g0=g0):
            m = jnp.max(hc, axis=0, keepdims=True)
            cur = pooled_ref[pl.ds(g0, 1), :]
            pooled_ref[pl.ds(g0, 1), :] = jnp.maximum(cur, m)

        @pl.when(s0 != s1)
        def _mixed(hc=hc, bbc=bbc, s0=s0, s1=s1, g0=g0):
            m0 = jnp.max(jnp.where(bbc == s0, hc, -jnp.inf), axis=0,
                         keepdims=True)
            m1 = jnp.max(jnp.where(bbc == s1, hc, -jnp.inf), axis=0,
                         keepdims=True)
            g1 = jnp.where(live, s1, _G)
            cur0 = pooled_ref[pl.ds(g0, 1), :]
            pooled_ref[pl.ds(g0, 1), :] = jnp.maximum(cur0, m0)
            cur1 = pooled_ref[pl.ds(g1, 1), :]
            pooled_ref[pl.ds(g1, 1), :] = jnp.maximum(cur1, m1)

            @pl.when(jnp.logical_and(live, s1 > s0 + 1))
            def _middles():
                def body(g, carry):
                    m = jnp.max(jnp.where(bbc == g, hc, -jnp.inf), axis=0,
                                keepdims=True)
                    cur = pooled_ref[pl.ds(g, 1), :]
                    pooled_ref[pl.ds(g, 1), :] = jnp.maximum(cur, m)
                    return carry

                jax.lax.fori_loop(s0 + 1, s1, body, 0)


def _ff_kernel(bounds_ref, xa_ref, xb_ref, batcha_ref, batchb_ref, w_enc_ref,
               b_enc_ref, w_out_ref, b_out_ref, out_ref, pooled_ref,
               buf_a, buf_b):
    i = pl.program_id(0)
    nb = pl.num_programs(0)

    @pl.when(i == 0)
    def _init():
        pooled_ref[...] = jnp.full_like(pooled_ref, -jnp.inf)

    buf_a[...] = jnp.dot(xa_ref[...], w_enc_ref[...],
                         preferred_element_type=jnp.float32)
    jb = jnp.maximum(2 * i - 1, 0)
    _epilogue(bounds_ref, batchb_ref, buf_b, pooled_ref, jb, i > 0)

    buf_b[...] = jnp.dot(xb_ref[...], w_enc_ref[...],
                         preferred_element_type=jnp.float32)
    ja = jnp.minimum(2 * i, _NB - 1)
    _epilogue(bounds_ref, batcha_ref, buf_a, pooled_ref, ja, i < nb - 1)

    @pl.when(i == nb - 1)
    def _finish():
        pooled = jnp.maximum(pooled_ref[0:_G, :] + b_enc_ref[...], 0.0)
        logits = (
            jnp.dot(pooled, w_out_ref[...],
                    preferred_element_type=jnp.float32)
            + b_out_ref[...]
        )
        mx = jnp.max(logits, axis=1, keepdims=True)
        sh = logits - mx
        lse = jnp.log(jnp.sum(jnp.exp(sh), axis=1, keepdims=True))
        out_ref[...] = sh - lse


def kernel(x, batch, W_enc, b_enc, W_out, b_out):
    batch = batch.astype(jnp.int32)
    batch_col = batch.reshape(_N, 1)
    starts = jnp.arange(_NB * _NC, dtype=jnp.int32) * _W
    bounds = jnp.stack([batch[starts], batch[starts + _W - 1]], axis=1)

    grid_spec = pltpu.PrefetchScalarGridSpec(
        num_scalar_prefetch=1,
        grid=(_NB2 + 1,),
        in_specs=[
            pl.BlockSpec((_B, _D),
                         lambda i, b: (jnp.minimum(2 * i, _NB - 1), 0)),
            pl.BlockSpec((_B, _D),
                         lambda i, b: (jnp.minimum(2 * i + 1, _NB - 1), 0)),
            pl.BlockSpec((_B, 1),
                         lambda i, b: (jnp.minimum(2 * i, _NB - 1), 0)),
            pl.BlockSpec((_B, 1),
                         lambda i, b: (jnp.maximum(2 * i - 1, 0), 0)),
            pl.BlockSpec((_D, _D), lambda i, b: (0, 0)),
            pl.BlockSpec((1, _D), lambda i, b: (0, 0)),
            pl.BlockSpec((_D, _C), lambda i, b: (0, 0)),
            pl.BlockSpec((1, _C), lambda i, b: (0, 0)),
        ],
        out_specs=pl.BlockSpec((_G, _C), lambda i, b: (0, 0)),
        scratch_shapes=[pltpu.VMEM((_G + 8, _D), jnp.float32),
                        pltpu.VMEM((_B, _D), jnp.float32),
                        pltpu.VMEM((_B, _D), jnp.float32)],
    )

    return pl.pallas_call(
        _ff_kernel,
        grid_spec=grid_spec,
        out_shape=jax.ShapeDtypeStruct((_G, _C), jnp.float32),
        compiler_params=pltpu.CompilerParams(
            dimension_semantics=("arbitrary",),
        ),
    )(bounds, x, x, batch_col, batch_col, W_enc, b_enc.reshape(1, _D), W_out,
      b_out.reshape(1, _C))


# speedup vs baseline: 1.0572x; 1.0097x over previous
import jax
import jax.numpy as jnp
from jax.experimental import pallas as pl
from jax.experimental.pallas import tpu as pltpu

_N = 50000
_D = 512
_G = 128
_C = 128
_B = 1000
_NB = _N // _B
_NB2 = _NB // 2
_NC = 5
_W = _B // _NC


def _epilogue(bounds_ref, batch_ref, h_ref, pooled_ref, jblk, live):
    bb = batch_ref[...]
    for c in range(_NC):
        hc = h_ref[c * _W:(c + 1) * _W, :]
        bbc = bb[c * _W:(c + 1) * _W, :]
        idx = jblk * _NC + c
        s0 = bounds_ref[idx, 0]
        s1 = bounds_ref[idx, 1]
        g0 = jnp.where(live, s0, _G)

        @pl.when(s0 == s1)
        def _pure(hc=hc, g0=g0):
            m = jnp.max(hc, axis=0, keepdims=True)
            cur = pooled_ref[pl.ds(g0, 1), :]
            pooled_ref[pl.ds(g0, 1), :] = jnp.maximum(cur, m)

        @pl.when(s0 != s1)
        def _mixed(hc=hc, bbc=bbc, s0=s0, s1=s1, g0=g0):
            m0 = jnp.max(jnp.where(bbc == s0, hc, -jnp.inf), axis=0,
                         keepdims=True)
            m1 = jnp.max(jnp.where(bbc == s1, hc, -jnp.inf), axis=0,
                         keepdims=True)
            g1 = jnp.where(live, s1, _G)
            cur0 = pooled_ref[pl.ds(g0, 1), :]
            pooled_ref[pl.ds(g0, 1), :] = jnp.maximum(cur0, m0)
            cur1 = pooled_ref[pl.ds(g1, 1), :]
            pooled_ref[pl.ds(g1, 1), :] = jnp.maximum(cur1, m1)

            @pl.when(jnp.logical_and(live, s1 > s0 + 1))
            def _middles():
                def body(g, carry):
                    m = jnp.max(jnp.where(bbc == g, hc, -jnp.inf), axis=0,
                                keepdims=True)
                    cur = pooled_ref[pl.ds(g, 1), :]
                    pooled_ref[pl.ds(g, 1), :] = jnp.maximum(cur, m)
                    return carry

                jax.lax.fori_loop(s0 + 1, s1, body, 0)


def _ff_kernel(bounds_ref, xa_ref, xb_ref, batcha_ref, batchb_ref, batchc_ref,
               w_enc_ref, b_enc_ref, w_out_ref, b_out_ref, out_ref, pooled_ref,
               buf_a, buf_b):
    i = pl.program_id(0)
    nb = pl.num_programs(0)

    @pl.when(i == 0)
    def _init():
        pooled_ref[...] = jnp.full_like(pooled_ref, -jnp.inf)

    buf_a[...] = jnp.dot(xa_ref[...], w_enc_ref[...],
                         preferred_element_type=jnp.float32)
    jb = jnp.maximum(2 * i - 1, 0)
    _epilogue(bounds_ref, batchb_ref, buf_b, pooled_ref, jb, i > 0)

    buf_b[...] = jnp.dot(xb_ref[...], w_enc_ref[...],
                         preferred_element_type=jnp.float32)
    _epilogue(bounds_ref, batcha_ref, buf_a, pooled_ref, 2 * i, i >= 0)

    @pl.when(i == nb - 1)
    def _finish():
        _epilogue(bounds_ref, batchc_ref, buf_b, pooled_ref,
                  jnp.int32(_NB - 1), i >= 0)
        pooled = jnp.maximum(pooled_ref[0:_G, :] + b_enc_ref[...], 0.0)
        logits = (
            jnp.dot(pooled, w_out_ref[...],
                    preferred_element_type=jnp.float32)
            + b_out_ref[...]
        )
        mx = jnp.max(logits, axis=1, keepdims=True)
        sh = logits - mx
        lse = jnp.log(jnp.sum(jnp.exp(sh), axis=1, keepdims=True))
        out_ref[...] = sh - lse


def kernel(x, batch, W_enc, b_enc, W_out, b_out):
    batch = batch.astype(jnp.int32)
    batch_col = batch.reshape(_N, 1)
    starts = jnp.arange(_NB * _NC, dtype=jnp.int32) * _W
    bounds = jnp.stack([batch[starts], batch[starts + _W - 1]], axis=1)

    grid_spec = pltpu.PrefetchScalarGridSpec(
        num_scalar_prefetch=1,
        grid=(_NB2,),
        in_specs=[
            pl.BlockSpec((_B, _D), lambda i, b: (2 * i, 0)),
            pl.BlockSpec((_B, _D), lambda i, b: (2 * i + 1, 0)),
            pl.BlockSpec((_B, 1), lambda i, b: (2 * i, 0)),
            pl.BlockSpec((_B, 1),
                         lambda i, b: (jnp.maximum(2 * i - 1, 0), 0)),
            pl.BlockSpec((_B, 1),
                         lambda i, b: (jnp.minimum(2 * i + 1, _NB - 1), 0)),
            pl.BlockSpec((_D, _D), lambda i, b: (0, 0)),
            pl.BlockSpec((1, _D), lambda i, b: (0, 0)),
            pl.BlockSpec((_D, _C), lambda i, b: (0, 0)),
            pl.BlockSpec((1, _C), lambda i, b: (0, 0)),
        ],
        out_specs=pl.BlockSpec((_G, _C), lambda i, b: (0, 0)),
        scratch_shapes=[pltpu.VMEM((_G + 8, _D), jnp.float32),
                        pltpu.VMEM((_B, _D), jnp.float32),
                        pltpu.VMEM((_B, _D), jnp.float32)],
    )

    return pl.pallas_call(
        _ff_kernel,
        grid_spec=grid_spec,
        out_shape=jax.ShapeDtypeStruct((_G, _C), jnp.float32),
        compiler_params=pltpu.CompilerParams(
            dimension_semantics=("arbitrary",),
        ),
    )(bounds, x, x, batch_col, batch_col, batch_col, W_enc,
      b_enc.reshape(1, _D), W_out, b_out.reshape(1, _C))
